# baseline (speedup 1.0000x reference)
import jax
import jax.numpy as jnp
from jax.experimental import pallas as pl
from jax.experimental.pallas import tpu as pltpu


_VMEM_LIMIT_BYTES = 48 * 1024 * 1024


def _round_up(x, m):
    return ((x + m - 1) // m) * m


def _pad2d(x, shape, dtype):
    if x.shape == tuple(shape):
        return x.astype(dtype)
    out = jnp.zeros(shape, dtype)
    return out.at[: x.shape[0], : x.shape[1]].set(x.astype(dtype))


def _layer1_kernel(a_ref, x_ref, w1_ref, b1_ref, w2_ref, h_ref, xw2_ref,
                   xw1_scr):
    @pl.when(pl.program_id(1) == 0)
    def _():
        xw1_scr[...] = jnp.dot(
            x_ref[...].astype(jnp.bfloat16), w1_ref[...],
            preferred_element_type=jnp.float32,
        ).astype(xw1_scr.dtype)

    acc = jnp.dot(a_ref[...], xw1_scr[...], preferred_element_type=jnp.float32)
    h_bf16 = jnp.maximum(acc + b1_ref[...], 0.0).astype(jnp.bfloat16)
    h_ref[...] = h_bf16.astype(h_ref.dtype)
    xw2_ref[...] = jnp.dot(
        h_bf16, w2_ref[...], preferred_element_type=jnp.float32
    ).astype(xw2_ref.dtype)


def _layer2_kernel(a_ref, xw2_ref, b2_ref, o_ref):
    acc = jnp.dot(a_ref[...], xw2_ref[...], preferred_element_type=jnp.float32)
    o_ref[...] = (acc + b2_ref[...]).astype(o_ref.dtype)


def _layer1(a_bf16, x_bf16, w1_bf16, b1_f32, w2_bf16, *, tm):
    n = a_bf16.shape[0]
    f = x_bf16.shape[1]
    h = w1_bf16.shape[1]
    c = w2_bf16.shape[1]
    nk = n // tm // 2
    cost = pl.CostEstimate(
        flops=2 * n * n * h + 2 * (2 * n * f * h) + 2 * n * h * c,
        transcendentals=0,
        bytes_accessed=n * n * 2 + n * f * 4 + n * h * 4 + n * c * 2,
    )
    return pl.pallas_call(
        _layer1_kernel,
        out_shape=(
            jax.ShapeDtypeStruct((n, h), jnp.float32),
            jax.ShapeDtypeStruct((n, c), jnp.bfloat16),
        ),
        grid=(2, nk),
        in_specs=[
            pl.BlockSpec((tm, n), lambda i, k: (i * (n // tm // 2) + k, 0)),
            pl.BlockSpec((n, f), lambda i, k: (0, 0)),
            pl.BlockSpec((f, h), lambda i, k: (0, 0)),
            pl.BlockSpec((1, h), lambda i, k: (0, 0)),
            pl.BlockSpec((h, c), lambda i, k: (0, 0)),
        ],
        out_specs=(
            pl.BlockSpec((tm, h), lambda i, k: (i * (n // tm // 2) + k, 0)),
            pl.BlockSpec((tm, c), lambda i, k: (i * (n // tm // 2) + k, 0)),
        ),
        scratch_shapes=[pltpu.VMEM((n, h), jnp.bfloat16)],
        compiler_params=pltpu.CompilerParams(
            dimension_semantics=("parallel", "arbitrary"),
            vmem_limit_bytes=_VMEM_LIMIT_BYTES),
        cost_estimate=cost,
    )(a_bf16, x_bf16, w1_bf16, b1_f32, w2_bf16)


def _layer2(a_bf16, xw2_bf16, b2_f32, *, tm):
    n = a_bf16.shape[0]
    c = xw2_bf16.shape[1]
    cost = pl.CostEstimate(
        flops=2 * n * n * c,
        transcendentals=0,
        bytes_accessed=n * n * 2 + n * c * 2 + n * c * 4,
    )
    return pl.pallas_call(
        _layer2_kernel,
        out_shape=jax.ShapeDtypeStruct((n, c), jnp.float32),
        grid=(n // tm,),
        in_specs=[
            pl.BlockSpec((tm, n), lambda i: (i, 0)),
            pl.BlockSpec((n, c), lambda i: (0, 0)),
            pl.BlockSpec((1, c), lambda i: (0, 0)),
        ],
        out_specs=pl.BlockSpec((tm, c), lambda i: (i, 0)),
        compiler_params=pltpu.CompilerParams(
            dimension_semantics=("parallel",),
            vmem_limit_bytes=_VMEM_LIMIT_BYTES),
        cost_estimate=cost,
    )(a_bf16, xw2_bf16, b2_f32)


def _xw_kernel(x_ref, w_ref, o_ref):
    o_ref[...] = jnp.dot(
        x_ref[...], w_ref[...], preferred_element_type=jnp.float32
    ).astype(o_ref.dtype)


def _feature_transform(x_bf16, w_bf16, *, tm):
    n, f = x_bf16.shape
    h = w_bf16.shape[1]
    return pl.pallas_call(
        _xw_kernel,
        out_shape=jax.ShapeDtypeStruct((n, h), jnp.bfloat16),
        grid=(n // tm,),
        in_specs=[
            pl.BlockSpec((tm, f), lambda i: (i, 0)),
            pl.BlockSpec((f, h), lambda i: (0, 0)),
        ],
        out_specs=pl.BlockSpec((tm, h), lambda i: (i, 0)),
        compiler_params=pltpu.CompilerParams(
            dimension_semantics=("parallel",),
            vmem_limit_bytes=_VMEM_LIMIT_BYTES),
    )(x_bf16, w_bf16)


def _layer1_kernel_simple(a_ref, xw1_ref, b1_ref, w2_ref, h_ref, xw2_ref):
    acc = jnp.dot(a_ref[...], xw1_ref[...], preferred_element_type=jnp.float32)
    h_bf16 = jnp.maximum(acc + b1_ref[...], 0.0).astype(jnp.bfloat16)
    h_ref[...] = h_bf16.astype(h_ref.dtype)
    xw2_ref[...] = jnp.dot(
        h_bf16, w2_ref[...], preferred_element_type=jnp.float32
    ).astype(xw2_ref.dtype)


def _layer1_simple(a_bf16, xw1_bf16, b1_f32, w2_bf16, *, tm):
    n = a_bf16.shape[0]
    h = xw1_bf16.shape[1]
    c = w2_bf16.shape[1]
    return pl.pallas_call(
        _layer1_kernel_simple,
        out_shape=(
            jax.ShapeDtypeStruct((n, h), jnp.float32),
            jax.ShapeDtypeStruct((n, c), jnp.bfloat16),
        ),
        grid=(n // tm,),
        in_specs=[
            pl.BlockSpec((tm, n), lambda i: (i, 0)),
            pl.BlockSpec((n, h), lambda i: (0, 0)),
            pl.BlockSpec((1, h), lambda i: (0, 0)),
            pl.BlockSpec((h, c), lambda i: (0, 0)),
        ],
        out_specs=(
            pl.BlockSpec((tm, h), lambda i: (i, 0)),
            pl.BlockSpec((tm, c), lambda i: (i, 0)),
        ),
        compiler_params=pltpu.CompilerParams(
            dimension_semantics=("parallel",),
            vmem_limit_bytes=_VMEM_LIMIT_BYTES),
    )(a_bf16, xw1_bf16, b1_f32, w2_bf16)


def kernel(w1, b1, w2, b2, x, a_hat_pad):
    n, f = x.shape
    n_pad = a_hat_pad.shape[0]
    hidden = w1.shape[1]
    num_classes = w2.shape[1]

    f_pad = _round_up(f, 128)
    h_pad = _round_up(hidden, 128)
    c_pad = _round_up(num_classes, 64)
    tm = 1024 if n_pad % 1024 == 0 else (256 if n_pad % 256 == 0 else 128)

    a_bf16 = a_hat_pad.astype(jnp.bfloat16)
    x_p = _pad2d(x, (n_pad, f_pad), jnp.bfloat16)
    w1_p = _pad2d(w1, (f_pad, h_pad), jnp.bfloat16)
    b1_p = _pad2d(b1.reshape(1, -1), (1, h_pad), jnp.float32)
    w2_p = _pad2d(w2, (h_pad, c_pad), jnp.bfloat16)
    b2_p = _pad2d(b2.reshape(1, -1), (1, c_pad), jnp.float32)

    if (n_pad // tm) % 2 == 0:
        xf_p = _pad2d(x, (n_pad, f_pad), jnp.float32)
        h_full, xw2 = _layer1(a_bf16, xf_p, w1_p, b1_p, w2_p, tm=tm)
    else:
        xw1 = _feature_transform(x_p, w1_p, tm=tm)
        h_full, xw2 = _layer1_simple(a_bf16, xw1, b1_p, w2_p, tm=tm)
    z_full = _layer2(a_bf16, xw2, b2_p, tm=tm)

    return h_full[:n, :hidden], z_full[:n, :num_classes]


# speedup vs baseline: 1.0124x; 1.0124x over previous
import jax
import jax.numpy as jnp
from jax.experimental import pallas as pl
from jax.experimental.pallas import tpu as pltpu


_VMEM_LIMIT_BYTES = 48 * 1024 * 1024


def _round_up(x, m):
    return ((x + m - 1) // m) * m


def _pad2d(x, shape, dtype):
    if x.shape == tuple(shape):
        return x.astype(dtype)
    out = jnp.zeros(shape, dtype)
    return out.at[: x.shape[0], : x.shape[1]].set(x.astype(dtype))


def _layer1_kernel(a_ref, x_ref, w1_ref, b1_ref, w2_ref, h_ref, xw2_ref,
                   xw1_scr):
    @pl.when(pl.program_id(1) == 0)
    def _():
        xw1_scr[...] = jnp.dot(
            x_ref[...].astype(jnp.bfloat16), w1_ref[...],
            preferred_element_type=jnp.float32,
        ).astype(xw1_scr.dtype)

    acc = jnp.dot(a_ref[...], xw1_scr[...], preferred_element_type=jnp.float32)
    h_bf16 = jnp.maximum(acc + b1_ref[...], 0.0).astype(jnp.bfloat16)
    h_ref[...] = h_bf16.astype(h_ref.dtype)
    xw2_ref[...] = jnp.dot(
        h_bf16, w2_ref[...], preferred_element_type=jnp.float32
    ).astype(xw2_ref.dtype)


def _layer2_kernel(a_ref, xw2_ref, b2_ref, o_ref):
    acc = jnp.dot(a_ref[...], xw2_ref[...], preferred_element_type=jnp.float32)
    o_ref[...] = (acc + b2_ref[...]).astype(o_ref.dtype)


def _layer1(a_bf16, x_bf16, w1_bf16, b1_f32, w2_bf16, *, tm):
    n = a_bf16.shape[0]
    f = x_bf16.shape[1]
    h = w1_bf16.shape[1]
    c = w2_bf16.shape[1]
    nk = n // tm // 2
    cost = pl.CostEstimate(
        flops=2 * n * n * h + 2 * (2 * n * f * h) + 2 * n * h * c,
        transcendentals=0,
        bytes_accessed=n * n * 2 + n * f * 4 + n * h * 4 + n * c * 2,
    )
    return pl.pallas_call(
        _layer1_kernel,
        out_shape=(
            jax.ShapeDtypeStruct((n, h), jnp.float32),
            jax.ShapeDtypeStruct((n, c), jnp.bfloat16),
        ),
        grid=(2, nk),
        in_specs=[
            pl.BlockSpec((tm, n), lambda i, k: (i * (n // tm // 2) + k, 0)),
            pl.BlockSpec((n, f), lambda i, k: (0, 0)),
            pl.BlockSpec((f, h), lambda i, k: (0, 0)),
            pl.BlockSpec((1, h), lambda i, k: (0, 0)),
            pl.BlockSpec((h, c), lambda i, k: (0, 0)),
        ],
        out_specs=(
            pl.BlockSpec((tm, h), lambda i, k: (i * (n // tm // 2) + k, 0)),
            pl.BlockSpec((tm, c), lambda i, k: (i * (n // tm // 2) + k, 0)),
        ),
        scratch_shapes=[pltpu.VMEM((n, h), jnp.bfloat16)],
        compiler_params=pltpu.CompilerParams(
            dimension_semantics=("parallel", "arbitrary"),
            vmem_limit_bytes=_VMEM_LIMIT_BYTES),
    )(a_bf16, x_bf16, w1_bf16, b1_f32, w2_bf16)


def _layer2(a_bf16, xw2_bf16, b2_f32, *, tm):
    n = a_bf16.shape[0]
    c = xw2_bf16.shape[1]
    cost = pl.CostEstimate(
        flops=2 * n * n * c,
        transcendentals=0,
        bytes_accessed=n * n * 2 + n * c * 2 + n * c * 4,
    )
    return pl.pallas_call(
        _layer2_kernel,
        out_shape=jax.ShapeDtypeStruct((n, c), jnp.float32),
        grid=(n // tm,),
        in_specs=[
            pl.BlockSpec((tm, n), lambda i: (i, 0)),
            pl.BlockSpec((n, c), lambda i: (0, 0)),
            pl.BlockSpec((1, c), lambda i: (0, 0)),
        ],
        out_specs=pl.BlockSpec((tm, c), lambda i: (i, 0)),
        compiler_params=pltpu.CompilerParams(
            dimension_semantics=("parallel",),
            vmem_limit_bytes=_VMEM_LIMIT_BYTES),
    )(a_bf16, xw2_bf16, b2_f32)


def _xw_kernel(x_ref, w_ref, o_ref):
    o_ref[...] = jnp.dot(
        x_ref[...], w_ref[...], preferred_element_type=jnp.float32
    ).astype(o_ref.dtype)


def _feature_transform(x_bf16, w_bf16, *, tm):
    n, f = x_bf16.shape
    h = w_bf16.shape[1]
    return pl.pallas_call(
        _xw_kernel,
        out_shape=jax.ShapeDtypeStruct((n, h), jnp.bfloat16),
        grid=(n // tm,),
        in_specs=[
            pl.BlockSpec((tm, f), lambda i: (i, 0)),
            pl.BlockSpec((f, h), lambda i: (0, 0)),
        ],
        out_specs=pl.BlockSpec((tm, h), lambda i: (i, 0)),
        compiler_params=pltpu.CompilerParams(
            dimension_semantics=("parallel",),
            vmem_limit_bytes=_VMEM_LIMIT_BYTES),
    )(x_bf16, w_bf16)


def _layer1_kernel_simple(a_ref, xw1_ref, b1_ref, w2_ref, h_ref, xw2_ref):
    acc = jnp.dot(a_ref[...], xw1_ref[...], preferred_element_type=jnp.float32)
    h_bf16 = jnp.maximum(acc + b1_ref[...], 0.0).astype(jnp.bfloat16)
    h_ref[...] = h_bf16.astype(h_ref.dtype)
    xw2_ref[...] = jnp.dot(
        h_bf16, w2_ref[...], preferred_element_type=jnp.float32
    ).astype(xw2_ref.dtype)


def _layer1_simple(a_bf16, xw1_bf16, b1_f32, w2_bf16, *, tm):
    n = a_bf16.shape[0]
    h = xw1_bf16.shape[1]
    c = w2_bf16.shape[1]
    return pl.pallas_call(
        _layer1_kernel_simple,
        out_shape=(
            jax.ShapeDtypeStruct((n, h), jnp.float32),
            jax.ShapeDtypeStruct((n, c), jnp.bfloat16),
        ),
        grid=(n // tm,),
        in_specs=[
            pl.BlockSpec((tm, n), lambda i: (i, 0)),
            pl.BlockSpec((n, h), lambda i: (0, 0)),
            pl.BlockSpec((1, h), lambda i: (0, 0)),
            pl.BlockSpec((h, c), lambda i: (0, 0)),
        ],
        out_specs=(
            pl.BlockSpec((tm, h), lambda i: (i, 0)),
            pl.BlockSpec((tm, c), lambda i: (i, 0)),
        ),
        compiler_params=pltpu.CompilerParams(
            dimension_semantics=("parallel",),
            vmem_limit_bytes=_VMEM_LIMIT_BYTES),
    )(a_bf16, xw1_bf16, b1_f32, w2_bf16)


def kernel(w1, b1, w2, b2, x, a_hat_pad):
    n, f = x.shape
    n_pad = a_hat_pad.shape[0]
    hidden = w1.shape[1]
    num_classes = w2.shape[1]

    f_pad = _round_up(f, 128)
    h_pad = _round_up(hidden, 128)
    c_pad = _round_up(num_classes, 64)
    tm = 1024 if n_pad % 1024 == 0 else (256 if n_pad % 256 == 0 else 128)

    a_bf16 = a_hat_pad.astype(jnp.bfloat16)
    x_p = _pad2d(x, (n_pad, f_pad), jnp.bfloat16)
    w1_p = _pad2d(w1, (f_pad, h_pad), jnp.bfloat16)
    b1_p = _pad2d(b1.reshape(1, -1), (1, h_pad), jnp.float32)
    w2_p = _pad2d(w2, (h_pad, c_pad), jnp.bfloat16)
    b2_p = _pad2d(b2.reshape(1, -1), (1, c_pad), jnp.float32)

    if (n_pad // tm) % 2 == 0:
        xf_p = _pad2d(x, (n_pad, f_pad), jnp.float32)
        h_full, xw2 = _layer1(a_bf16, xf_p, w1_p, b1_p, w2_p, tm=tm)
    else:
        xw1 = _feature_transform(x_p, w1_p, tm=tm)
        h_full, xw2 = _layer1_simple(a_bf16, xw1, b1_p, w2_p, tm=tm)
    z_full = _layer2(a_bf16, xw2, b2_p, tm=tm)

    return h_full[:n, :hidden], z_full[:n, :num_classes]
